# baseline (speedup 1.0000x reference)
import functools

import jax
import jax.numpy as jnp
import numpy as np
from jax import lax
from jax.experimental import pallas as pl
from jax.experimental.pallas import tpu as pltpu
from jax.experimental.pallas import tpu_sc as plsc

_TIME_STEPS = 1000
_BETA_START = 0.0001
_BETA_END = 0.02

_betas = np.linspace(_BETA_START, _BETA_END, _TIME_STEPS, dtype=np.float64)
_alphas_cumprod = np.cumprod(1.0 - _betas)
_TABLE_A = np.broadcast_to(
    np.sqrt(_alphas_cumprod).astype(np.float32)[:, None], (_TIME_STEPS, 128)
).copy()
_TABLE_C = np.broadcast_to(
    np.sqrt(1.0 - _alphas_cumprod).astype(np.float32)[:, None],
    (_TIME_STEPS, 128),
).copy()

_ROW = 3 * 128 * 128
_CH = 16384
_UNROLL = 8


def _make_sc_fma(batch):
    info = plsc.get_sparse_core_info()
    nc, ns = info.num_cores, info.num_subcores
    nw = nc * ns
    b_per_w = batch // nw
    span = b_per_w * _ROW
    n_chunks = span // _CH
    chunks_per_row = _ROW // _CH
    mesh = plsc.VectorSubcoreMesh(core_axis_name="c", subcore_axis_name="s")

    @functools.partial(
        pl.kernel,
        mesh=mesh,
        out_type=jax.ShapeDtypeStruct((batch * _ROW,), jnp.float32),
        scratch_types=[
            pltpu.VMEM((b_per_w,), jnp.int32),
            pltpu.VMEM((b_per_w, 128), jnp.float32),
            pltpu.VMEM((b_per_w, 128), jnp.float32),
            pltpu.VMEM((_CH,), jnp.float32),
            pltpu.VMEM((_CH,), jnp.float32),
            pltpu.VMEM((_CH,), jnp.float32),
            pltpu.VMEM((_CH,), jnp.float32),
            pltpu.VMEM((_CH,), jnp.float32),
            pltpu.VMEM((_CH,), jnp.float32),
            pltpu.SemaphoreType.DMA,
            pltpu.SemaphoreType.DMA,
            pltpu.SemaphoreType.DMA,
            pltpu.SemaphoreType.DMA,
            pltpu.SemaphoreType.DMA,
            pltpu.SemaphoreType.DMA,
            pltpu.SemaphoreType.DMA,
        ],
    )
    def fma_k(xf, nf, idx_hbm, tab_a, tab_c, of,
              idx_v, rows_va, rows_vc, xb0, xb1, nb0, nb1, ob0, ob1,
              sg, sx0, sx1, sn0, sn1, so0, so1):
        wid = lax.axis_index("s") * nc + lax.axis_index("c")
        base = wid * b_per_w
        woff = wid * span

        pltpu.sync_copy(idx_hbm.at[pl.ds(base, b_per_w)], idx_v)
        pltpu.async_copy(tab_a.at[idx_v], rows_va, sg).wait()
        pltpu.async_copy(tab_c.at[idx_v], rows_vc, sg).wait()

        xb = [xb0, xb1]
        nb = [nb0, nb1]
        ob = [ob0, ob1]
        sx = [sx0, sx1]
        sn = [sn0, sn1]
        so = [so0, so1]
        cpx = {}
        cpn = {}
        cpo = {}

        def start_in(k):
            bi = k % 2
            off = woff + k * _CH
            cpx[k] = pltpu.async_copy(xf.at[pl.ds(off, _CH)], xb[bi], sx[bi])
            cpn[k] = pltpu.async_copy(nf.at[pl.ds(off, _CH)], nb[bi], sn[bi])

        start_in(0)
        a = c = None
        for k in range(n_chunks):
            bi = k % 2
            if k % chunks_per_row == 0:
                r = k // chunks_per_row
                a = rows_va[r, pl.ds(0, 16)]
                c = rows_vc[r, pl.ds(0, 16)]
            if k + 1 < n_chunks:
                start_in(k + 1)
            cpx[k].wait()
            cpn[k].wait()
            if k >= 2:
                cpo[k - 2].wait()
            xr, nr, orr = xb[bi], nb[bi], ob[bi]
            aa, cc = a, c

            def inner(j, _):
                s0 = j * (16 * _UNROLL)
                for u in range(_UNROLL):
                    s = s0 + u * 16
                    orr[pl.ds(s, 16)] = (
                        aa * xr[pl.ds(s, 16)] + cc * nr[pl.ds(s, 16)]
                    )
                return 0

            lax.fori_loop(0, _CH // (16 * _UNROLL), inner, 0)
            off = woff + k * _CH
            cpo[k] = pltpu.async_copy(orr, of.at[pl.ds(off, _CH)], so[bi])
        cpo[n_chunks - 2].wait()
        cpo[n_chunks - 1].wait()

    return fma_k


@jax.jit
def kernel(x, time, noise):
    b = x.shape[0]
    xf = x.reshape(-1)
    nf = noise.reshape(-1)
    tab_a = jnp.asarray(_TABLE_A)
    tab_c = jnp.asarray(_TABLE_C)
    out = _make_sc_fma(b)(xf, nf, time, tab_a, tab_c)
    return out.reshape(x.shape)


# speedup vs baseline: 1.6455x; 1.6455x over previous
import functools

import jax
import jax.numpy as jnp
import numpy as np
from jax.experimental import pallas as pl
from jax.experimental.pallas import tpu as pltpu

_TIME_STEPS = 1000
_BETA_START = 0.0001
_BETA_END = 0.02

_betas = np.linspace(_BETA_START, _BETA_END, _TIME_STEPS, dtype=np.float64)
_alphas_cumprod = np.cumprod(1.0 - _betas)
_TABLE = np.concatenate(
    [np.sqrt(_alphas_cumprod), np.sqrt(1.0 - _alphas_cumprod)]
).astype(np.float32)

_BB = 16


def _fma_body(time_ref, tab_ref, x_ref, n_ref, o_ref):
    g = pl.program_id(0)
    for i in range(_BB):
        t = time_ref[g * _BB + i]
        a = tab_ref[t]
        c = tab_ref[_TIME_STEPS + t]
        o_ref[i] = a * x_ref[i] + c * n_ref[i]


@jax.jit
def kernel(x, time, noise):
    b, ch, h, w = x.shape
    rows = ch * h * w // 128
    x3 = x.reshape(b, rows, 128)
    n3 = noise.reshape(b, rows, 128)
    tab = jnp.asarray(_TABLE)

    grid = b // _BB
    spec = pl.BlockSpec((_BB, rows, 128), lambda g, *_: (g, 0, 0))
    out = pl.pallas_call(
        _fma_body,
        grid_spec=pltpu.PrefetchScalarGridSpec(
            num_scalar_prefetch=2,
            grid=(grid,),
            in_specs=[spec, spec],
            out_specs=spec,
        ),
        compiler_params=pltpu.CompilerParams(
            dimension_semantics=("parallel",),
        ),
        out_shape=jax.ShapeDtypeStruct((b, rows, 128), jnp.float32),
    )(time, tab, x3, n3)
    return out.reshape(x.shape)
